# baseline (speedup 1.0000x reference)
import functools
import jax
import jax.numpy as jnp
from jax import lax
from jax.experimental import pallas as pl
from jax.experimental.pallas import tpu as pltpu
from jax.experimental.pallas import tpu_sc as plsc

N = 10000
E = 320000
D = 128
B = 64
C = 10
RATIO = 0.8

NP = 10240
NB = NP // 128
NC, NS = 2, 16
NW = NC * NS
EPW = E // NW
KE = 80
NCH = EPW // KE
RPT = NP // NS

def _sc_body(x_hbm, src_hbm, dst_hbm, zeros_hbm, out_hbm,
             src_v, dst_v, rows_v, agg_sh, sem):
    cid = lax.axis_index("c")
    sid = lax.axis_index("s")
    wid = sid * NC + cid
    pltpu.sync_copy(zeros_hbm.at[pl.ds(sid * RPT, RPT)],
                    agg_sh.at[pl.ds(sid * RPT, RPT)])
    pltpu.sync_copy(src_hbm.at[wid], src_v)
    pltpu.sync_copy(dst_hbm.at[wid], dst_v)
    plsc.subcore_barrier()

    def body(i, carry):
        pltpu.async_copy(x_hbm.at[src_v.at[i]], rows_v, sem).wait()
        pltpu.sync_copy(rows_v, agg_sh.at[dst_v.at[i]], add=True)
        return carry

    lax.fori_loop(0, NCH, body, 0)
    plsc.subcore_barrier()
    pltpu.sync_copy(agg_sh.at[pl.ds(sid * RPT, RPT)],
                    out_hbm.at[pl.ds(cid * NP + sid * RPT, RPT)])


@functools.cache
def _get_sc_kernel():
    mesh = plsc.VectorSubcoreMesh(core_axis_name="c", subcore_axis_name="s")
    return functools.partial(
        pl.kernel,
        mesh=mesh,
        out_type=jax.ShapeDtypeStruct((NC * NP, D), jnp.float32),
        scratch_types=[
            pltpu.VMEM((NCH, KE), jnp.int32),
            pltpu.VMEM((NCH, KE), jnp.int32),
            pltpu.VMEM((KE, D), jnp.float32),
            pltpu.VMEM_SHARED((NP, D), jnp.float32),
            pltpu.SemaphoreType.DMA,
        ],
    )(_sc_body)


def _edge_agg(x_pad, src_w, dst_w, zeros):
    return _get_sc_kernel()(x_pad, src_w, dst_w, zeros)


def _rows_valid(i):
    gidx = i * 128 + lax.broadcasted_iota(jnp.int32, (128, 1), 0)
    return gidx < N


def _score(h, p_ref):
    p = p_ref[...]
    pn = jnp.sqrt(jnp.sum(p * p)) + 1e-16
    return jnp.tanh(jnp.dot(h, p, preferred_element_type=jnp.float32) / pn)


def _gin_body(x_ref, a0_ref, a1_ref, w1_ref, b1_ref, w2_ref, b2_ref,
              eps_ref, p_ref, xo_ref, sc_ref):
    i = pl.program_id(0)
    valid = _rows_valid(i)
    agg = jnp.where(valid, a0_ref[...] + a1_ref[...], 0.0)
    h = (1.0 + eps_ref[0, 0]) * x_ref[...] + agg
    h = jnp.dot(h, w1_ref[...], preferred_element_type=jnp.float32) + b1_ref[...]
    h = jnp.dot(h, w2_ref[...], preferred_element_type=jnp.float32) + b2_ref[...]
    x = jnp.maximum(h, 0.0)
    xo_ref[...] = x
    sc_ref[...] = _score(x, p_ref)


def _conv_body(x_ref, a0_ref, a1_ref, wr_ref, we_ref, b_ref, m_ref, p_ref,
               xo_ref, sc_ref):
    i = pl.program_id(0)
    valid = _rows_valid(i)
    agg = jnp.where(valid, a0_ref[...] + a1_ref[...], 0.0)
    h = (jnp.dot(x_ref[...], wr_ref[...], preferred_element_type=jnp.float32)
         + jnp.dot(agg, we_ref[...], preferred_element_type=jnp.float32)
         + b_ref[...])
    x = jnp.where(m_ref[...] > 0.0, jnp.maximum(h, 0.0), 0.0)
    xo_ref[...] = x
    sc_ref[...] = _score(x, p_ref)


_BLK = pl.BlockSpec((128, 128), lambda i: (i, 0))
_COL = pl.BlockSpec((128, 1), lambda i: (i, 0))
_FULLW = pl.BlockSpec((128, 128), lambda i: (0, 0))
_ROWV = pl.BlockSpec((1, 128), lambda i: (0, 0))
_PCOL = pl.BlockSpec((128, 1), lambda i: (0, 0))
_ONE = pl.BlockSpec((1, 1), lambda i: (0, 0))


def _agg_specs():
    return [pl.BlockSpec((128, 128), lambda i: (i, 0)),
            pl.BlockSpec((128, 128), lambda i: (NB + i, 0))]


def _gin_dense(x_pad, agg2, w1, b1, w2, b2, eps, p):
    return pl.pallas_call(
        _gin_body,
        grid=(NB,),
        in_specs=[_BLK] + _agg_specs() + [_FULLW, _ROWV, _FULLW, _ROWV, _ONE, _PCOL],
        out_specs=[_BLK, _COL],
        out_shape=[jax.ShapeDtypeStruct((NP, D), jnp.float32),
                   jax.ShapeDtypeStruct((NP, 1), jnp.float32)],
    )(x_pad, agg2, agg2, w1, b1.reshape(1, D), w2, b2.reshape(1, D),
      eps.reshape(1, 1), p.reshape(D, 1))


def _conv_dense(x_pad, agg2, wr, we, b, mask_col, p):
    return pl.pallas_call(
        _conv_body,
        grid=(NB,),
        in_specs=[_BLK] + _agg_specs() + [_FULLW, _FULLW, _ROWV, _COL, _PCOL],
        out_specs=[_BLK, _COL],
        out_shape=[jax.ShapeDtypeStruct((NP, D), jnp.float32),
                   jax.ShapeDtypeStruct((NP, 1), jnp.float32)],
    )(x_pad, agg2, agg2, wr, we, b.reshape(1, D), mask_col, p.reshape(D, 1))


def _rankpool_body(x_ref, sc_col_ref, sc_row_ref, bat_col_ref, bat_row_ref,
                   al_col_ref, al_row_ref,
                   xo_ref, nm_ref, pmax_ref, psum_ref, pcnt_ref):
    i = pl.program_id(0)

    @pl.when(i == 0)
    def _init():
        pmax_ref[...] = jnp.full((B, D), -jnp.inf, jnp.float32)
        psum_ref[...] = jnp.zeros((B, D), jnp.float32)
        pcnt_ref[...] = jnp.zeros((B, 128), jnp.float32)

    s_i = sc_col_ref[...]
    b_i = bat_col_ref[...]
    a_i = al_col_ref[...] > 0.0
    gid_i = i * 128 + lax.broadcasted_iota(jnp.int32, (128, 1), 0)

    def jstep(j, carry):
        rank, cnt = carry
        s_j = sc_row_ref[pl.ds(j, 1), :]
        b_j = bat_row_ref[pl.ds(j, 1), :]
        a_j = al_row_ref[pl.ds(j, 1), :] > 0.0
        gid_j = j * 128 + lax.broadcasted_iota(jnp.int32, (1, 128), 1)
        same = (b_j == b_i) & a_j
        better = (s_j > s_i) | ((s_j == s_i) & (gid_j < gid_i))
        rank = rank + jnp.sum(jnp.where(same & better, 1.0, 0.0), axis=1,
                              keepdims=True)
        cnt = cnt + jnp.sum(jnp.where(same, 1.0, 0.0), axis=1, keepdims=True)
        return rank, cnt

    z = jnp.zeros((128, 1), jnp.float32)
    rank, cnt = lax.fori_loop(0, NB, jstep, (z, z))
    k = jnp.ceil(RATIO * cnt)
    keep = a_i & (rank < k)
    xnew = jnp.where(keep, x_ref[...] * s_i, 0.0)
    xo_ref[...] = xnew
    nm_ref[...] = jnp.where(keep, 1.0, 0.0)

    def gstep(g, carry):
        gm = keep & (b_i == g)
        xm = jnp.where(gm, xnew, -1e9)
        gmax = jnp.max(xm, axis=0, keepdims=True)
        gsum = jnp.sum(jnp.where(gm, xnew, 0.0), axis=0, keepdims=True)
        gcnt = jnp.sum(jnp.where(gm, 1.0, 0.0))
        any_g = jnp.max(jnp.where(gm, 1.0, 0.0))
        gmax = jnp.where(any_g > 0.0, gmax, -jnp.inf)
        pmax_ref[pl.ds(g, 1), :] = jnp.maximum(pmax_ref[pl.ds(g, 1), :], gmax)
        psum_ref[pl.ds(g, 1), :] = psum_ref[pl.ds(g, 1), :] + gsum
        pcnt_ref[pl.ds(g, 1), :] = pcnt_ref[pl.ds(g, 1), :] + gcnt
        return carry

    lax.fori_loop(0, B, gstep, 0)


def _rankpool(x_pad, sc_col, bat_col, bat_row, al_col, al_row):
    sc_row = sc_col.reshape(NB, 128)
    return pl.pallas_call(
        _rankpool_body,
        grid=(NB,),
        in_specs=[_BLK, _COL,
                  pl.BlockSpec((NB, 128), lambda i: (0, 0)),
                  _COL,
                  pl.BlockSpec((NB, 128), lambda i: (0, 0)),
                  _COL,
                  pl.BlockSpec((NB, 128), lambda i: (0, 0))],
        out_specs=[_BLK, _COL,
                   pl.BlockSpec((B, D), lambda i: (0, 0)),
                   pl.BlockSpec((B, D), lambda i: (0, 0)),
                   pl.BlockSpec((B, 128), lambda i: (0, 0))],
        out_shape=[jax.ShapeDtypeStruct((NP, D), jnp.float32),
                   jax.ShapeDtypeStruct((NP, 1), jnp.float32),
                   jax.ShapeDtypeStruct((B, D), jnp.float32),
                   jax.ShapeDtypeStruct((B, D), jnp.float32),
                   jax.ShapeDtypeStruct((B, 128), jnp.float32)],
    )(x_pad, sc_col, sc_row, bat_col, bat_row, al_col, al_row)


def _head_body(m1_ref, s1_ref, c1_ref, m2_ref, s2_ref, c2_ref,
               m3_ref, s3_ref, c3_ref, w1a_ref, w1b_ref, b1_ref,
               w2_ref, b2_ref, w3_ref, b3_ref, o_ref):
    gmax = m1_ref[...] + m2_ref[...] + m3_ref[...]
    gap = (s1_ref[...] / jnp.maximum(c1_ref[...], 1.0)
           + s2_ref[...] / jnp.maximum(c2_ref[...], 1.0)
           + s3_ref[...] / jnp.maximum(c3_ref[...], 1.0))
    h = (jnp.dot(gmax, w1a_ref[...], preferred_element_type=jnp.float32)
         + jnp.dot(gap, w1b_ref[...], preferred_element_type=jnp.float32)
         + b1_ref[...])
    h = jnp.maximum(h, 0.0)
    h = jnp.dot(h, w2_ref[...], preferred_element_type=jnp.float32) + b2_ref[...]
    h = jnp.maximum(h, 0.0)
    z = jnp.dot(h, w3_ref[...], preferred_element_type=jnp.float32) + b3_ref[...]
    zmax = jnp.max(z, axis=1, keepdims=True)
    zs = z - zmax
    lse = jnp.log(jnp.sum(jnp.exp(zs), axis=1, keepdims=True))
    o_ref[...] = zs - lse


def _head(pools, w1, b1, w2, b2, w3, b3):
    ins = list(pools) + [w1[:D], w1[D:], b1.reshape(1, D), w2,
                         b2.reshape(1, 64), w3, b3.reshape(1, C)]
    return pl.pallas_call(
        _head_body,
        in_specs=[pl.BlockSpec(a.shape, lambda i: (0, 0)) for a in ins],
        out_specs=pl.BlockSpec((B, C), lambda i: (0, 0)),
        out_shape=jax.ShapeDtypeStruct((B, C), jnp.float32),
        grid=(1,),
    )(*ins)


def kernel(x, W_gin1, b_gin1, W_gin2, b_gin2, eps, p1, W2_root, W2_rel, b2, p2,
           W3_root, W3_rel, b3, p3, W_lin1, b_lin1, W_lin2, b_lin2, W_lin3, b_lin3,
           edge_index, batch):
    x_pad = jnp.concatenate([x, jnp.zeros((NP - N, D), jnp.float32)], axis=0)
    src_w = edge_index[0].astype(jnp.int32).reshape(NW, NCH, KE)
    dst_w = edge_index[1].astype(jnp.int32).reshape(NW, NCH, KE)
    zeros = jnp.zeros((NP, D), jnp.float32)
    bat_pad = jnp.concatenate(
        [batch.astype(jnp.int32), jnp.full((NP - N,), B + 63, jnp.int32)])
    bat_col = bat_pad.reshape(NP, 1)
    bat_row = bat_pad.reshape(NB, 128)
    ones_col = jnp.concatenate(
        [jnp.ones((N, 1), jnp.float32), jnp.zeros((NP - N, 1), jnp.float32)])

    agg = _edge_agg(x_pad, src_w, dst_w, zeros)
    x_pad, sc = _gin_dense(x_pad, agg, W_gin1, b_gin1, W_gin2, b_gin2, eps, p1)
    x_pad, m_col, pm1, ps1, pc1 = _rankpool(
        x_pad, sc, bat_col, bat_row, ones_col, ones_col.reshape(NB, 128))

    agg = _edge_agg(x_pad, src_w, dst_w, zeros)
    x_pad, sc = _conv_dense(x_pad, agg, W2_root, W2_rel, b2, m_col, p2)
    x_pad, m_col, pm2, ps2, pc2 = _rankpool(
        x_pad, sc, bat_col, bat_row, m_col, m_col.reshape(NB, 128))

    agg = _edge_agg(x_pad, src_w, dst_w, zeros)
    x_pad, sc = _conv_dense(x_pad, agg, W3_root, W3_rel, b3, m_col, p3)
    x_pad, m_col, pm3, ps3, pc3 = _rankpool(
        x_pad, sc, bat_col, bat_row, m_col, m_col.reshape(NB, 128))

    return _head((pm1, ps1, pc1, pm2, ps2, pc2, pm3, ps3, pc3),
                 W_lin1, b_lin1, W_lin2, b_lin2, W_lin3, b_lin3)
